# baseline (speedup 1.0000x reference)
import functools

import jax
import jax.numpy as jnp
from jax import lax
from jax.experimental import pallas as pl
from jax.experimental.pallas import tpu as pltpu
from jax.experimental.pallas import tpu_sc as plsc

NUM_CLASSES = 64
N_SUPPORT = 8192
N_QUERY = 16384
D_FEAT = 512

NC = 2
NS = 16
NW = NC * NS
ROWS_PER_TILE = N_SUPPORT // NW
SC_BATCH = 64
NB = ROWS_PER_TILE // SC_BATCH
CNT_W = 128


def _sc_segment_sums(support_features, support_labels):
    zeros_sum = jnp.zeros((NUM_CLASSES * D_FEAT,), jnp.float32)
    zeros_cnt = jnp.zeros((NUM_CLASSES * CNT_W,), jnp.float32)

    mesh = plsc.VectorSubcoreMesh(core_axis_name="c", subcore_axis_name="s",
                                  num_cores=NC, num_subcores=NS)

    @functools.partial(
        pl.kernel,
        out_type=(
            jax.ShapeDtypeStruct((NW, NUM_CLASSES * D_FEAT), jnp.float32),
            jax.ShapeDtypeStruct((NW, NUM_CLASSES * CNT_W), jnp.float32),
        ),
        mesh=mesh,
        scratch_types=[
            pltpu.VMEM((SC_BATCH, D_FEAT), jnp.float32),
            pltpu.VMEM((SC_BATCH, D_FEAT), jnp.float32),
            pltpu.VMEM((SC_BATCH,), jnp.int32),
            pltpu.VMEM((SC_BATCH,), jnp.int32),
            pltpu.VMEM((NUM_CLASSES * D_FEAT,), jnp.float32),
            pltpu.VMEM((NUM_CLASSES * CNT_W,), jnp.float32),
            pltpu.SemaphoreType.DMA,
            pltpu.SemaphoreType.DMA,
            pltpu.SemaphoreType.DMA,
            pltpu.SemaphoreType.DMA,
            pltpu.SemaphoreType.DMA,
        ],
        compiler_params=pltpu.CompilerParams(use_tc_tiling_on_sc=True,
                                             needs_layout_passes=False),
    )
    def seg_kernel(feat_hbm, lab_hbm, zsum_hbm, zcnt_hbm,
                   out_sums, out_cnts,
                   rb0, rb1, lb0, lb1, acc_sum, acc_cnt,
                   sf0, sf1, sl0, sl1, sz):
        c = lax.axis_index("c")
        s = lax.axis_index("s")

        col0 = lax.iota(jnp.int32, 16)
        ones_v = jnp.full((16,), 1.0, jnp.float32)

        wid = s * NC + c
        base = wid * ROWS_PER_TILE
        bufs = [(rb0, lb0, sf0, sl0), (rb1, lb1, sf1, sl1)]

        dz0 = pltpu.async_copy(zsum_hbm, acc_sum, sz)
        dz1 = pltpu.async_copy(zcnt_hbm, acc_cnt, sz)

        def start(b, slot):
            rb, lbuf, sf, sl = bufs[slot]
            off = base + b * SC_BATCH
            pltpu.async_copy(feat_hbm.at[pl.ds(off, SC_BATCH)], rb, sf)
            pltpu.async_copy(lab_hbm.at[pl.ds(off, SC_BATCH)], lbuf, sl)

        def waitbuf(slot):
            rb, lbuf, sf, sl = bufs[slot]
            pltpu.make_async_copy(
                feat_hbm.at[pl.ds(0, SC_BATCH)], rb, sf).wait()
            pltpu.make_async_copy(
                lab_hbm.at[pl.ds(0, SC_BATCH)], lbuf, sl).wait()

        def compute(rb, lbuf):
            def group(i, _):
                labs16 = lbuf[pl.ds(i * 16, 16)]
                v0 = jnp.take(labs16, jnp.full((16,), 0, jnp.int32))
                v15 = jnp.take(labs16, jnp.full((16,), 15, jnp.int32))
                plsc.addupdate_scatter(
                    acc_cnt, [labs16 * CNT_W + col0], ones_v)
                uniform = jnp.all(labs16 == v0)

                def tree(vs):
                    while len(vs) > 1:
                        vs = [vs[a] + vs[a + 1] for a in range(0, len(vs), 2)]
                    return vs[0]

                def fast_path():
                    base0 = jnp.min(labs16) * D_FEAT

                    def chunk(kk, _):
                        sl16 = pl.ds(kk * 16, 16)
                        s = tree([rb[i * 16 + j, sl16] for j in range(16)])
                        dst = pl.ds(base0 + kk * 16, 16)
                        acc_sum[dst] = acc_sum[dst] + s
                        return 0
                    lax.fori_loop(0, D_FEAT // 16, chunk, 0, unroll=2)

                def boundary_path():
                    two = jnp.all((labs16 == v0) | (labs16 == v15))

                    def two_class():
                        masks = [jnp.take(labs16, jnp.full((16,), j, jnp.int32))
                                 == v0 for j in range(16)]
                        base_a = jnp.min(labs16) * D_FEAT
                        base_b = jnp.max(labs16) * D_FEAT
                        zero_v = jnp.zeros((16,), jnp.float32)

                        def chunk2(kk, _):
                            sl16 = pl.ds(kk * 16, 16)
                            rows = [rb[i * 16 + j, sl16] for j in range(16)]
                            total = tree(list(rows))
                            part_a = tree([
                                jnp.where(masks[j], rows[j], zero_v)
                                for j in range(16)])
                            dst_a = pl.ds(base_a + kk * 16, 16)
                            acc_sum[dst_a] = acc_sum[dst_a] + part_a
                            dst_b = pl.ds(base_b + kk * 16, 16)
                            acc_sum[dst_b] = acc_sum[dst_b] + (total - part_a)
                            return 0
                        lax.fori_loop(0, D_FEAT // 16, chunk2, 0, unroll=2)

                    def per_row():
                        def row(j, _):
                            labj = jnp.take(labs16,
                                            jnp.full((16,), j, jnp.int32))
                            rowbase = labj * D_FEAT + col0

                            def chunk_s(kk, _):
                                chunkv = rb[i * 16 + j, pl.ds(kk * 16, 16)]
                                plsc.addupdate_scatter(
                                    acc_sum, [rowbase + (kk * 16)], chunkv)
                                return 0
                            lax.fori_loop(0, D_FEAT // 16, chunk_s, 0,
                                          unroll=4)
                            return 0
                        lax.fori_loop(0, 16, row, 0, unroll=False)

                    lax.cond(two, two_class, per_row)

                lax.cond(uniform, fast_path, boundary_path)
                return 0
            lax.fori_loop(0, SC_BATCH // 16, group, 0, unroll=False)

        start(0, 0)
        start(1, 1)
        dz0.wait()
        dz1.wait()

        def pair(t, _):
            waitbuf(0)
            compute(rb0, lb0)

            @pl.when(t < NB // 2 - 1)
            def _():
                start(2 * t + 2, 0)

            waitbuf(1)
            compute(rb1, lb1)

            @pl.when(t < NB // 2 - 1)
            def _():
                start(2 * t + 3, 1)
            return 0
        lax.fori_loop(0, NB // 2, pair, 0, unroll=False)

        pltpu.sync_copy(acc_sum, out_sums.at[wid])
        pltpu.sync_copy(acc_cnt, out_cnts.at[wid])

    return seg_kernel(support_features, support_labels, zeros_sum, zeros_cnt)


BQ = 2048
Q2R = 8


def _q2_body(q_ref, out_ref):
    q = q_ref[...]
    ones_m = jnp.ones((Q2R, D_FEAT), jnp.float32)
    out_ref[...] = lax.dot_general(ones_m, q * q, (((1,), (1,)), ((), ())),
                                   preferred_element_type=jnp.float32)


def _tc_body(psums_ref, pcnts_ref, q_ref, q2_ref, out_ref, proto_ref, p2_ref):
    @pl.when(pl.program_id(0) == 0)
    def _():
        sums = jnp.sum(psums_ref[...], axis=0).reshape(
            NUM_CLASSES, D_FEAT)
        cnt_t = jnp.sum(pcnts_ref[...], axis=0).reshape(
            NUM_CLASSES, CNT_W)
        cnts = jnp.sum(cnt_t, axis=1, keepdims=True)
        proto = sums / cnts
        proto_ref[...] = -2.0 * proto
        p2_ref[...] = jnp.sum(proto * proto, axis=1,
                              keepdims=True)

    q = q_ref[...]
    qpm2t = lax.dot_general(proto_ref[...], q, (((1,), (1,)), ((), ())),
                            preferred_element_type=jnp.float32)
    d2t = q2_ref[0:1, :] + p2_ref[...] + qpm2t
    out_ref[...] = -jnp.sqrt(jnp.maximum(d2t, 0.0))


def kernel(support_features, support_labels, query_features):
    psums, pcnts = _sc_segment_sums(support_features, support_labels)
    grid = (N_QUERY // BQ,)
    q2 = pl.pallas_call(
        _q2_body,
        grid=grid,
        in_specs=[pl.BlockSpec((BQ, D_FEAT), lambda i: (i, 0))],
        out_specs=pl.BlockSpec((Q2R, BQ), lambda i: (0, i)),
        out_shape=jax.ShapeDtypeStruct((Q2R, N_QUERY), jnp.float32),
        compiler_params=pltpu.CompilerParams(
            dimension_semantics=("arbitrary",),
        ),
    )(query_features)
    out_t = pl.pallas_call(
        _tc_body,
        grid=grid,
        in_specs=[
            pl.BlockSpec((NW, NUM_CLASSES * D_FEAT), lambda i: (0, 0)),
            pl.BlockSpec((NW, NUM_CLASSES * CNT_W), lambda i: (0, 0)),
            pl.BlockSpec((BQ, D_FEAT), lambda i: (i, 0)),
            pl.BlockSpec((Q2R, BQ), lambda i: (0, i)),
        ],
        out_specs=pl.BlockSpec((NUM_CLASSES, BQ), lambda i: (0, i)),
        out_shape=jax.ShapeDtypeStruct((NUM_CLASSES, N_QUERY), jnp.float32),
        scratch_shapes=[
            pltpu.VMEM((NUM_CLASSES, D_FEAT), jnp.float32),
            pltpu.VMEM((NUM_CLASSES, 1), jnp.float32),
        ],
        compiler_params=pltpu.CompilerParams(
            dimension_semantics=("arbitrary",),
        ),
    )(psums, pcnts, query_features, q2)
    return out_t.T


# speedup vs baseline: 1.0918x; 1.0918x over previous
import functools

import jax
import jax.numpy as jnp
from jax import lax
from jax.experimental import pallas as pl
from jax.experimental.pallas import tpu as pltpu
from jax.experimental.pallas import tpu_sc as plsc

NUM_CLASSES = 64
N_SUPPORT = 8192
N_QUERY = 16384
D_FEAT = 512

NC = 2
NS = 16
NW = NC * NS
ROWS_PER_TILE = N_SUPPORT // NW
SC_BATCH = 64
NB = ROWS_PER_TILE // SC_BATCH
CNT_W = 128


def _sc_segment_sums(support_features, support_labels):
    zeros_sum = jnp.zeros((NUM_CLASSES * D_FEAT,), jnp.float32)
    zeros_cnt = jnp.zeros((NUM_CLASSES * CNT_W,), jnp.float32)

    mesh = plsc.VectorSubcoreMesh(core_axis_name="c", subcore_axis_name="s",
                                  num_cores=NC, num_subcores=NS)

    @functools.partial(
        pl.kernel,
        out_type=(
            jax.ShapeDtypeStruct((NW, NUM_CLASSES * D_FEAT), jnp.float32),
            jax.ShapeDtypeStruct((NW, NUM_CLASSES * CNT_W), jnp.float32),
        ),
        mesh=mesh,
        scratch_types=[
            pltpu.VMEM((SC_BATCH, D_FEAT), jnp.float32),
            pltpu.VMEM((SC_BATCH, D_FEAT), jnp.float32),
            pltpu.VMEM((SC_BATCH,), jnp.int32),
            pltpu.VMEM((SC_BATCH,), jnp.int32),
            pltpu.VMEM((NUM_CLASSES * D_FEAT,), jnp.float32),
            pltpu.VMEM((NUM_CLASSES * CNT_W,), jnp.float32),
            pltpu.SemaphoreType.DMA,
            pltpu.SemaphoreType.DMA,
            pltpu.SemaphoreType.DMA,
            pltpu.SemaphoreType.DMA,
            pltpu.SemaphoreType.DMA,
        ],
        compiler_params=pltpu.CompilerParams(use_tc_tiling_on_sc=True,
                                             needs_layout_passes=False),
    )
    def seg_kernel(feat_hbm, lab_hbm, zsum_hbm, zcnt_hbm,
                   out_sums, out_cnts,
                   rb0, rb1, lb0, lb1, acc_sum, acc_cnt,
                   sf0, sf1, sl0, sl1, sz):
        c = lax.axis_index("c")
        s = lax.axis_index("s")

        col0 = lax.iota(jnp.int32, 16)
        ones_v = jnp.full((16,), 1.0, jnp.float32)

        wid = s * NC + c
        base = wid * ROWS_PER_TILE
        bufs = [(rb0, lb0, sf0, sl0), (rb1, lb1, sf1, sl1)]

        dz0 = pltpu.async_copy(zsum_hbm, acc_sum, sz)
        dz1 = pltpu.async_copy(zcnt_hbm, acc_cnt, sz)

        def start(b, slot):
            rb, lbuf, sf, sl = bufs[slot]
            off = base + b * SC_BATCH
            pltpu.async_copy(feat_hbm.at[pl.ds(off, SC_BATCH)], rb, sf)
            pltpu.async_copy(lab_hbm.at[pl.ds(off, SC_BATCH)], lbuf, sl)

        def waitbuf(slot):
            rb, lbuf, sf, sl = bufs[slot]
            pltpu.make_async_copy(
                feat_hbm.at[pl.ds(0, SC_BATCH)], rb, sf).wait()
            pltpu.make_async_copy(
                lab_hbm.at[pl.ds(0, SC_BATCH)], lbuf, sl).wait()

        def compute(rb, lbuf):
            def group(i, _):
                labs16 = lbuf[pl.ds(i * 16, 16)]
                v0 = jnp.take(labs16, jnp.full((16,), 0, jnp.int32))
                v15 = jnp.take(labs16, jnp.full((16,), 15, jnp.int32))
                plsc.addupdate_scatter(
                    acc_cnt, [labs16 * CNT_W + col0], ones_v)
                uniform = jnp.all(labs16 == v0)

                def tree(vs):
                    while len(vs) > 1:
                        vs = [vs[a] + vs[a + 1] for a in range(0, len(vs), 2)]
                    return vs[0]

                def fast_path():
                    base0 = jnp.min(labs16) * D_FEAT

                    @functools.partial(
                        plsc.parallel_loop, 0, D_FEAT // 16, unroll=2)
                    def _chunks(kk):
                        sl16 = pl.ds(kk * 16, 16)
                        s = tree([rb[i * 16 + j, sl16] for j in range(16)])
                        dst = pl.ds(base0 + kk * 16, 16)
                        acc_sum[dst] = acc_sum[dst] + s

                def boundary_path():
                    two = jnp.all((labs16 == v0) | (labs16 == v15))

                    def two_class():
                        masks = [jnp.take(labs16, jnp.full((16,), j, jnp.int32))
                                 == v0 for j in range(16)]
                        base_a = jnp.min(labs16) * D_FEAT
                        base_b = jnp.max(labs16) * D_FEAT
                        zero_v = jnp.zeros((16,), jnp.float32)

                        def chunk2(kk, _):
                            sl16 = pl.ds(kk * 16, 16)
                            rows = [rb[i * 16 + j, sl16] for j in range(16)]
                            total = tree(list(rows))
                            part_a = tree([
                                jnp.where(masks[j], rows[j], zero_v)
                                for j in range(16)])
                            dst_a = pl.ds(base_a + kk * 16, 16)
                            acc_sum[dst_a] = acc_sum[dst_a] + part_a
                            dst_b = pl.ds(base_b + kk * 16, 16)
                            acc_sum[dst_b] = acc_sum[dst_b] + (total - part_a)
                            return 0
                        lax.fori_loop(0, D_FEAT // 16, chunk2, 0, unroll=2)

                    def per_row():
                        def row(j, _):
                            labj = jnp.take(labs16,
                                            jnp.full((16,), j, jnp.int32))
                            rowbase = labj * D_FEAT + col0

                            def chunk_s(kk, _):
                                chunkv = rb[i * 16 + j, pl.ds(kk * 16, 16)]
                                plsc.addupdate_scatter(
                                    acc_sum, [rowbase + (kk * 16)], chunkv)
                                return 0
                            lax.fori_loop(0, D_FEAT // 16, chunk_s, 0,
                                          unroll=4)
                            return 0
                        lax.fori_loop(0, 16, row, 0, unroll=False)

                    lax.cond(two, two_class, per_row)

                lax.cond(uniform, fast_path, boundary_path)
                return 0
            lax.fori_loop(0, SC_BATCH // 16, group, 0, unroll=False)

        start(0, 0)
        start(1, 1)
        dz0.wait()
        dz1.wait()

        def pair(t, _):
            waitbuf(0)
            compute(rb0, lb0)

            @pl.when(t < NB // 2 - 1)
            def _():
                start(2 * t + 2, 0)

            waitbuf(1)
            compute(rb1, lb1)

            @pl.when(t < NB // 2 - 1)
            def _():
                start(2 * t + 3, 1)
            return 0
        lax.fori_loop(0, NB // 2, pair, 0, unroll=False)

        pltpu.sync_copy(acc_sum, out_sums.at[wid])
        pltpu.sync_copy(acc_cnt, out_cnts.at[wid])

    return seg_kernel(support_features, support_labels, zeros_sum, zeros_cnt)


BQ = 2048
Q2R = 8


def _q2_body(q_ref, out_ref):
    q = q_ref[...]
    ones_m = jnp.ones((Q2R, D_FEAT), jnp.float32)
    out_ref[...] = lax.dot_general(ones_m, q * q, (((1,), (1,)), ((), ())),
                                   preferred_element_type=jnp.float32)


def _tc_body(psums_ref, pcnts_ref, q_ref, q2_ref, out_ref, proto_ref, p2_ref):
    @pl.when(pl.program_id(0) == 0)
    def _():
        sums = jnp.sum(psums_ref[...], axis=0).reshape(
            NUM_CLASSES, D_FEAT)
        cnt_t = jnp.sum(pcnts_ref[...], axis=0).reshape(
            NUM_CLASSES, CNT_W)
        cnts = jnp.sum(cnt_t, axis=1, keepdims=True)
        proto = sums / cnts
        proto_ref[...] = -2.0 * proto
        p2_ref[...] = jnp.sum(proto * proto, axis=1,
                              keepdims=True)

    q = q_ref[...]
    qpm2t = lax.dot_general(proto_ref[...], q, (((1,), (1,)), ((), ())),
                            preferred_element_type=jnp.float32)
    d2t = q2_ref[0:1, :] + p2_ref[...] + qpm2t
    out_ref[...] = -jnp.sqrt(jnp.maximum(d2t, 0.0))


def kernel(support_features, support_labels, query_features):
    psums, pcnts = _sc_segment_sums(support_features, support_labels)
    grid = (N_QUERY // BQ,)
    q2 = pl.pallas_call(
        _q2_body,
        grid=grid,
        in_specs=[pl.BlockSpec((BQ, D_FEAT), lambda i: (i, 0))],
        out_specs=pl.BlockSpec((Q2R, BQ), lambda i: (0, i)),
        out_shape=jax.ShapeDtypeStruct((Q2R, N_QUERY), jnp.float32),
        compiler_params=pltpu.CompilerParams(
            dimension_semantics=("arbitrary",),
        ),
    )(query_features)
    out_t = pl.pallas_call(
        _tc_body,
        grid=grid,
        in_specs=[
            pl.BlockSpec((NW, NUM_CLASSES * D_FEAT), lambda i: (0, 0)),
            pl.BlockSpec((NW, NUM_CLASSES * CNT_W), lambda i: (0, 0)),
            pl.BlockSpec((BQ, D_FEAT), lambda i: (i, 0)),
            pl.BlockSpec((Q2R, BQ), lambda i: (0, i)),
        ],
        out_specs=pl.BlockSpec((NUM_CLASSES, BQ), lambda i: (0, i)),
        out_shape=jax.ShapeDtypeStruct((NUM_CLASSES, N_QUERY), jnp.float32),
        scratch_shapes=[
            pltpu.VMEM((NUM_CLASSES, D_FEAT), jnp.float32),
            pltpu.VMEM((NUM_CLASSES, 1), jnp.float32),
        ],
        compiler_params=pltpu.CompilerParams(
            dimension_semantics=("arbitrary",),
        ),
    )(psums, pcnts, query_features, q2)
    return out_t.T
